# baseline (speedup 1.0000x reference)
import jax
import jax.numpy as jnp
from jax import lax
from jax.experimental import pallas as pl
from jax.experimental.pallas import tpu as pltpu
from jax.experimental.pallas import tpu_sc as plsc

NC = 2
NS = 16
NW = NC * NS

SEQ = 200
BATCH = 4096
DIM = 64
NBT = BATCH // 128
NTASK = SEQ * NBT
TPW = NTASK // NW
IPW = TPW * 128
CHUNK = 640
BPC = CHUNK // 128
NCH = IPW // CHUNK


def _emb_kernel(x_hbm, table_hbm, out_hbm, idx_v, rows, obs, sem_g, sem_s):
    wid = lax.axis_index("s") * NC + lax.axis_index("c")
    tbase = wid * TPW

    pltpu.sync_copy(x_hbm.at[pl.ds(wid * IPW, IPW)], idx_v)

    iota16 = lax.broadcasted_iota(jnp.int32, (16,), 0)

    def gather(c, b):
        return pltpu.async_copy(
            table_hbm.at[idx_v.at[pl.ds(c * CHUNK, CHUNK)]], rows[b], sem_g[b])

    def wait_gather(b):
        pltpu.make_async_copy(
            table_hbm.at[idx_v.at[pl.ds(0, CHUNK)]], rows[b], sem_g[b]).wait()

    def transpose(tb, b, ob):
        for bg in range(8):
            rowvec = iota16 + (tb * 128 + bg * 16)

            @plsc.parallel_loop(0, DIM, 1, unroll=8)
            def _(d):
                dt = lax.div(d, 8)
                dr = lax.rem(d, 8)
                val = plsc.load_gather(rows[b], [rowvec, jnp.full((16,), d, jnp.int32)])
                obs[ob][dt, dr, pl.ds(bg * 16, 16)] = val

    def store(t, ob):
        tt = tbase + t
        s = tt // NBT
        bt = tt % NBT
        return pltpu.async_copy(obs[ob], out_hbm.at[s, :, bt], sem_s[ob])

    def wait_store(ob):
        pltpu.make_async_copy(obs[ob], out_hbm.at[0, :, 0], sem_s[ob]).wait()

    gather(0, 0)
    gather(1, 1)

    def body(i, carry):
        for b in range(2):
            c = 2 * i + b
            wait_gather(b)
            for tb in range(BPC):
                ob = (b * BPC + tb) % 2

                @pl.when(10 * i + 5 * b + tb >= 2)
                def _():
                    wait_store(ob)

                transpose(tb, b, ob)
                store(c * BPC + tb, ob)

            @pl.when(c + 2 < NCH)
            def _():
                gather(c + 2, b)
        return carry

    lax.fori_loop(0, NCH // 2, body, 0)
    wait_store(0)
    wait_store(1)


def kernel(x, table):
    xT = jnp.transpose(x).reshape(-1)
    out5 = pl.kernel(
        _emb_kernel,
        out_type=jax.ShapeDtypeStruct((SEQ, 8, NBT, 8, 128), jnp.float32),
        mesh=plsc.VectorSubcoreMesh(core_axis_name="c", subcore_axis_name="s"),
        compiler_params=pltpu.CompilerParams(
            use_tc_tiling_on_sc=False, needs_layout_passes=False),
        scratch_types=[
            pltpu.VMEM((IPW,), jnp.int32),
            [pltpu.VMEM((CHUNK, DIM), jnp.float32) for _ in range(2)],
            [pltpu.VMEM((8, 8, 128), jnp.float32) for _ in range(2)],
            [pltpu.SemaphoreType.DMA for _ in range(2)],
            [pltpu.SemaphoreType.DMA for _ in range(2)],
        ],
    )(xT, table)
    return out5.transpose(2, 4, 0, 1, 3).reshape(BATCH, SEQ, DIM)


# speedup vs baseline: 1.4114x; 1.4114x over previous
import jax
import jax.numpy as jnp
from jax import lax
from jax.experimental import pallas as pl
from jax.experimental.pallas import tpu as pltpu
from jax.experimental.pallas import tpu_sc as plsc

NC = 2
NS = 16
NW = NC * NS

SEQ = 200
BATCH = 4096
DIM = 64
NBT = BATCH // 128
NTASK = SEQ * NBT
TPW = NTASK // NW
IPW = TPW * 128
CHUNK = 640
BPC = CHUNK // 128
NCH = IPW // CHUNK


def _emb_kernel(x_hbm, table_hbm, out_hbm, idx_v, rows, obs, sem_g, sem_s):
    wid = lax.axis_index("s") * NC + lax.axis_index("c")
    tbase = wid * TPW

    pltpu.sync_copy(x_hbm.at[pl.ds(wid * IPW, IPW)], idx_v)

    iota16 = lax.broadcasted_iota(jnp.int32, (16,), 0)

    def gather(c, b):
        return pltpu.async_copy(
            table_hbm.at[idx_v.at[pl.ds(c * CHUNK, CHUNK)]], rows[b], sem_g[b])

    def wait_gather(b):
        pltpu.make_async_copy(
            table_hbm.at[idx_v.at[pl.ds(0, CHUNK)]], rows[b], sem_g[b]).wait()

    def transpose(tb, b, ob):
        for bg in range(8):
            rowvec = iota16 + (tb * 128 + bg * 16)
            blvec = iota16 + (bg * 16)

            @plsc.parallel_loop(0, DIM, 1, unroll=8)
            def _(q):
                colvec = ((iota16 + q) & 15) + (q & ~15)
                val = plsc.load_gather(rows[b], [rowvec, colvec])
                dtv = lax.shift_right_logical(colvec, 3)
                drv = colvec & 7
                plsc.store_scatter(obs[ob], [dtv, drv, blvec], val)

    def store(t, ob):
        tt = tbase + t
        s = tt // NBT
        bt = tt % NBT
        return pltpu.async_copy(obs[ob], out_hbm.at[s, :, bt], sem_s[ob])

    def wait_store(ob):
        pltpu.make_async_copy(obs[ob], out_hbm.at[0, :, 0], sem_s[ob]).wait()

    gather(0, 0)
    gather(1, 1)

    def body(i, carry):
        for b in range(2):
            c = 2 * i + b
            wait_gather(b)
            for tb in range(BPC):
                ob = (b * BPC + tb) % 2

                @pl.when(10 * i + 5 * b + tb >= 2)
                def _():
                    wait_store(ob)

                transpose(tb, b, ob)
                store(c * BPC + tb, ob)

            @pl.when(c + 2 < NCH)
            def _():
                gather(c + 2, b)
        return carry

    lax.fori_loop(0, NCH // 2, body, 0)
    wait_store(0)
    wait_store(1)


def kernel(x, table):
    xT = jnp.transpose(x).reshape(-1)
    out5 = pl.kernel(
        _emb_kernel,
        out_type=jax.ShapeDtypeStruct((SEQ, 8, NBT, 8, 128), jnp.float32),
        mesh=plsc.VectorSubcoreMesh(core_axis_name="c", subcore_axis_name="s"),
        compiler_params=pltpu.CompilerParams(
            use_tc_tiling_on_sc=False, needs_layout_passes=False),
        scratch_types=[
            pltpu.VMEM((IPW,), jnp.int32),
            [pltpu.VMEM((CHUNK, DIM), jnp.float32) for _ in range(2)],
            [pltpu.VMEM((8, 8, 128), jnp.float32) for _ in range(2)],
            [pltpu.SemaphoreType.DMA for _ in range(2)],
            [pltpu.SemaphoreType.DMA for _ in range(2)],
        ],
    )(xT, table)
    return out5.transpose(2, 4, 0, 1, 3).reshape(BATCH, SEQ, DIM)
